# baseline (speedup 1.0000x reference)
import math

import jax
import jax.numpy as jnp
from jax.experimental import pallas as pl

B, S, D = 4, 2048, 2048
H, DH, FF = 16, 128, 8192
K = 512
KC = K - 1
T = B * K

_SQRT_DH = math.sqrt(float(DH))
_NEG = -1e30


def _ln_rows(x32, g_row, b_row):
    mu = jnp.mean(x32, axis=-1, keepdims=True)
    var = jnp.mean((x32 - mu) ** 2, axis=-1, keepdims=True)
    return (x32 - mu) * jax.lax.rsqrt(var + 1e-5) * g_row + b_row


def _route_gather_kernel(x_ref, wr_ref, wc_ref, g_ref, b_ref,
                         sel_ref, h_ref, selw_ref, slotc_ref):
    w_row = wr_ref[0]
    w_col = wc_ref[0]

    i_row = jax.lax.broadcasted_iota(jnp.int32, (1, S), 1)
    geq_row = jnp.zeros((1, S), jnp.int32)
    stab_row = jnp.zeros((1, S), jnp.int32)
    for c in range(8):
        chunk = w_col[c * 256:(c + 1) * 256, :]
        jidx = jax.lax.broadcasted_iota(jnp.int32, (256, 1), 0) + c * 256
        gt = chunk > w_row
        eq = chunk == w_row
        stab_row += jnp.sum((gt | (eq & (jidx < i_row))).astype(jnp.int32),
                            axis=0, keepdims=True)
        geq_row += jnp.sum((gt | eq).astype(jnp.int32), axis=0,
                           keepdims=True)
    mask_row = geq_row <= KC
    n_sel = jnp.sum(mask_row.astype(jnp.int32), axis=1, keepdims=True)
    slotc_ref[0] = jnp.where(mask_row, stab_row, KC)

    slots = jax.lax.broadcasted_iota(jnp.int32, (K, S), 0)
    toks = jax.lax.broadcasted_iota(jnp.int32, (K, S), 1)
    sel_onehot = (slots == stab_row) & mask_row
    dup_onehot = (slots >= n_sel) & (slots < KC) & (toks == S - 1)
    p = jnp.where(sel_onehot | dup_onehot, 1.0, 0.0)

    sel32 = jnp.dot(p.astype(jnp.bfloat16), x_ref[0].astype(jnp.bfloat16),
                    preferred_element_type=jnp.float32)
    sel_ref[0] = sel32.astype(jnp.bfloat16)
    selw_ref[0] = jnp.sum(p * w_row, axis=1, keepdims=True)
    h_ref[0] = _ln_rows(sel32, g_ref[:], b_ref[:]).astype(jnp.bfloat16)


def _qkv_kernel(h_ref, wq_ref, wk_ref, wv_ref, bq_ref, bk_ref, bv_ref,
                q_ref, k_ref, v_ref):
    h = h_ref[:]
    q_ref[:] = (jnp.dot(h, wq_ref[:].astype(jnp.bfloat16),
                        preferred_element_type=jnp.float32)
                + bq_ref[:]).astype(jnp.bfloat16)
    k_ref[:] = (jnp.dot(h, wk_ref[:].astype(jnp.bfloat16),
                        preferred_element_type=jnp.float32)
                + bk_ref[:]).astype(jnp.bfloat16)
    v_ref[:] = (jnp.dot(h, wv_ref[:].astype(jnp.bfloat16),
                        preferred_element_type=jnp.float32)
                + bv_ref[:]).astype(jnp.bfloat16)


_HPG = 4


def _attn_kernel(q_ref, k_ref, v_ref, w2_ref, out_ref, w2b_ref):
    w2b_ref[:] = w2_ref[:].astype(jnp.bfloat16)
    key_id = jax.lax.broadcasted_iota(jnp.int32, (K, K), 1)
    pad = key_id >= KC
    for t in range(_HPG):
        ds = pl.ds(t * DH, DH)
        s = jax.lax.dot_general(q_ref[:, ds], k_ref[:, ds],
                                (((1,), (1,)), ((), ())),
                                preferred_element_type=jnp.float32)
        s = s / _SQRT_DH
        s = jnp.where(pad, _NEG, s)
        m = jnp.max(s, axis=-1, keepdims=True)
        e = jnp.exp(s - m)
        a = e / jnp.sum(e, axis=-1, keepdims=True)
        out_ref[:, ds] = jnp.dot(a.astype(jnp.bfloat16), v_ref[:, ds],
                                 preferred_element_type=jnp.float32
                                 ).astype(jnp.bfloat16)


def _oproj_kernel(a_ref, wo_ref, bo_ref, sel_ref, g_ref, b_ref, w1_ref,
                  sel2_ref, h2_ref, w1b_ref):
    w1b_ref[:] = w1_ref[:].astype(jnp.bfloat16)
    o = jnp.dot(a_ref[:], wo_ref[:], preferred_element_type=jnp.float32)
    sel2 = sel_ref[:].astype(jnp.float32) + o + bo_ref[:]
    sel2_ref[:] = sel2.astype(jnp.bfloat16)
    h2_ref[:] = _ln_rows(sel2, g_ref[:], b_ref[:]).astype(jnp.bfloat16)


_NKFF = 16


def _mlp_kernel(h2_ref, w1_ref, b1_ref, w2_ref, y_ref):
    kk = pl.program_id(0)
    a = jnp.dot(h2_ref[:], w1_ref[:],
                preferred_element_type=jnp.float32) + b1_ref[:]
    g = jax.nn.gelu(a).astype(jnp.bfloat16)
    for n in range(4):
        ds = pl.ds(n * 512, 512)
        part = jnp.dot(g, w2_ref[:, ds],
                       preferred_element_type=jnp.float32)

        @pl.when(kk == 0)
        def _():
            y_ref[:, ds] = part

        @pl.when(kk > 0)
        def _():
            y_ref[:, ds] = y_ref[:, ds] + part


def _combine_kernel(slotc_ref, y_ref, sel2_ref, selw_ref, b2_ref, x_ref,
                    out_ref):
    yf = (sel2_ref[0].astype(jnp.float32) + y_ref[0] + b2_ref[:]
          ) * selw_ref[0]
    yb = yf.astype(jnp.bfloat16)
    slotc = slotc_ref[0]
    slot_id = jax.lax.broadcasted_iota(jnp.int32, (slotc.shape[0], K), 1)
    pt = jnp.where(slot_id == slotc, 1.0, 0.0).astype(jnp.bfloat16)
    ysel = jnp.dot(pt, yb, preferred_element_type=jnp.float32)
    out_ref[0] = jnp.where(slotc < KC, ysel, x_ref[0])


def kernel(x, attention_mask, position_ids, Wr, br, ln1_g, ln1_b,
           Wq, bq, Wk, bk, Wv, bv, Wo, bo, ln2_g, ln2_b, W1, b1, W2, b2):
    del attention_mask, position_ids
    f32, bf16 = jnp.float32, jnp.bfloat16

    w = (x @ Wr + br).squeeze(-1)
    w_row = w.reshape(B, 1, S)
    w_col = w.reshape(B, S, 1)

    g1, b1_, g2, b2_ = (ln1_g.reshape(1, D), ln1_b.reshape(1, D),
                        ln2_g.reshape(1, D), ln2_b.reshape(1, D))

    sel, h, selw, slotc = pl.pallas_call(
        _route_gather_kernel,
        grid=(B,),
        in_specs=[
            pl.BlockSpec((1, S, D), lambda i: (i, 0, 0)),
            pl.BlockSpec((1, 1, S), lambda i: (i, 0, 0)),
            pl.BlockSpec((1, S, 1), lambda i: (i, 0, 0)),
            pl.BlockSpec((1, D), lambda i: (0, 0)),
            pl.BlockSpec((1, D), lambda i: (0, 0)),
        ],
        out_specs=[
            pl.BlockSpec((1, K, D), lambda i: (i, 0, 0)),
            pl.BlockSpec((1, K, D), lambda i: (i, 0, 0)),
            pl.BlockSpec((1, K, 1), lambda i: (i, 0, 0)),
            pl.BlockSpec((1, 1, S), lambda i: (i, 0, 0)),
        ],
        out_shape=[
            jax.ShapeDtypeStruct((B, K, D), bf16),
            jax.ShapeDtypeStruct((B, K, D), bf16),
            jax.ShapeDtypeStruct((B, K, 1), f32),
            jax.ShapeDtypeStruct((B, 1, S), jnp.int32),
        ],
    )(x, w_row, w_col, g1, b1_)
    slotc = slotc.reshape(B, S, 1)

    h2d = h.reshape(T, D)
    sel2d = sel.reshape(T, D)

    BN = 256
    NB = D // BN
    q, kproj, v = pl.pallas_call(
        _qkv_kernel,
        grid=(NB,),
        in_specs=[
            pl.BlockSpec((T, D), lambda j: (0, 0)),
            pl.BlockSpec((D, BN), lambda j: (0, j)),
            pl.BlockSpec((D, BN), lambda j: (0, j)),
            pl.BlockSpec((D, BN), lambda j: (0, j)),
            pl.BlockSpec((1, BN), lambda j: (0, j)),
            pl.BlockSpec((1, BN), lambda j: (0, j)),
            pl.BlockSpec((1, BN), lambda j: (0, j)),
        ],
        out_specs=[
            pl.BlockSpec((T, BN), lambda j: (0, j)),
            pl.BlockSpec((T, BN), lambda j: (0, j)),
            pl.BlockSpec((T, BN), lambda j: (0, j)),
        ],
        out_shape=[
            jax.ShapeDtypeStruct((T, D), bf16),
            jax.ShapeDtypeStruct((T, D), bf16),
            jax.ShapeDtypeStruct((T, D), bf16),
        ],
    )(h2d, Wq, Wk, Wv, bq.reshape(1, D), bk.reshape(1, D), bv.reshape(1, D))

    NC = B * (H // _HPG)
    attn, w2_b = pl.pallas_call(
        _attn_kernel,
        grid=(B, H // _HPG),
        in_specs=[
            pl.BlockSpec((K, _HPG * DH), lambda b, hh: (b, hh)),
            pl.BlockSpec((K, _HPG * DH), lambda b, hh: (b, hh)),
            pl.BlockSpec((K, _HPG * DH), lambda b, hh: (b, hh)),
            pl.BlockSpec((FF // NC, D), lambda b, hh: (b * (H // _HPG) + hh,
                                                       0)),
        ],
        out_specs=[
            pl.BlockSpec((K, _HPG * DH), lambda b, hh: (b, hh)),
            pl.BlockSpec((FF // NC, D), lambda b, hh: (b * (H // _HPG) + hh,
                                                       0)),
        ],
        out_shape=[
            jax.ShapeDtypeStruct((T, D), bf16),
            jax.ShapeDtypeStruct((FF, D), bf16),
        ],
    )(q, kproj, v, W2)

    BM = 256
    ND1 = T // BM
    wo_b = Wo.astype(bf16)
    sel2, h2, w1_b = pl.pallas_call(
        _oproj_kernel,
        grid=(ND1,),
        in_specs=[
            pl.BlockSpec((BM, D), lambda i: (i, 0)),
            pl.BlockSpec((D, D), lambda i: (0, 0)),
            pl.BlockSpec((1, D), lambda i: (0, 0)),
            pl.BlockSpec((BM, D), lambda i: (i, 0)),
            pl.BlockSpec((1, D), lambda i: (0, 0)),
            pl.BlockSpec((1, D), lambda i: (0, 0)),
            pl.BlockSpec((D, FF // ND1), lambda i: (0, i)),
        ],
        out_specs=[
            pl.BlockSpec((BM, D), lambda i: (i, 0)),
            pl.BlockSpec((BM, D), lambda i: (i, 0)),
            pl.BlockSpec((D, FF // ND1), lambda i: (0, i)),
        ],
        out_shape=[
            jax.ShapeDtypeStruct((T, D), bf16),
            jax.ShapeDtypeStruct((T, D), bf16),
            jax.ShapeDtypeStruct((D, FF), bf16),
        ],
    )(attn, wo_b, bo.reshape(1, D), sel2d, g2, b2_, W1)

    BK = FF // _NKFF
    y = pl.pallas_call(
        _mlp_kernel,
        grid=(_NKFF,),
        in_specs=[
            pl.BlockSpec((T, D), lambda kk: (0, 0)),
            pl.BlockSpec((D, BK), lambda kk: (0, kk)),
            pl.BlockSpec((1, BK), lambda kk: (0, kk)),
            pl.BlockSpec((BK, D), lambda kk: (kk, 0)),
        ],
        out_specs=pl.BlockSpec((T, D), lambda kk: (0, 0)),
        out_shape=jax.ShapeDtypeStruct((T, D), f32),
    )(h2, w1_b, b1.reshape(1, FF), w2_b)

    y3 = y.reshape(B, K, D)
    sel23 = sel2.reshape(B, K, D)
    selw3 = selw

    ST = 512
    out = pl.pallas_call(
        _combine_kernel,
        grid=(B, S // ST),
        in_specs=[
            pl.BlockSpec((1, ST, 1), lambda b, t: (b, t, 0)),
            pl.BlockSpec((1, K, D), lambda b, t: (b, 0, 0)),
            pl.BlockSpec((1, K, D), lambda b, t: (b, 0, 0)),
            pl.BlockSpec((1, K, 1), lambda b, t: (b, 0, 0)),
            pl.BlockSpec((1, D), lambda b, t: (0, 0)),
            pl.BlockSpec((1, ST, D), lambda b, t: (b, t, 0)),
        ],
        out_specs=pl.BlockSpec((1, ST, D), lambda b, t: (b, t, 0)),
        out_shape=jax.ShapeDtypeStruct((B, S, D), f32),
    )(slotc, y3, sel23, selw3, b2.reshape(1, D), x)

    return out


# speedup vs baseline: 1.0580x; 1.0580x over previous
import math

import jax
import jax.numpy as jnp
from jax.experimental import pallas as pl

B, S, D = 4, 2048, 2048
H, DH, FF = 16, 128, 8192
K = 512
KC = K - 1
T = B * K

_SQRT_DH = math.sqrt(float(DH))
_NEG = -1e30


def _ln_rows(x32, g_row, b_row):
    mu = jnp.mean(x32, axis=-1, keepdims=True)
    var = jnp.mean((x32 - mu) ** 2, axis=-1, keepdims=True)
    return (x32 - mu) * jax.lax.rsqrt(var + 1e-5) * g_row + b_row


def _route_gather_kernel(x_ref, wr_ref, wc_ref, g_ref, b_ref,
                         sel_ref, h_ref, selw_ref, slotc_ref):
    w_row = wr_ref[0]
    w_col = wc_ref[0]

    i_row = jax.lax.broadcasted_iota(jnp.int32, (1, S), 1)
    geq_row = jnp.zeros((1, S), jnp.int32)
    stab_row = jnp.zeros((1, S), jnp.int32)
    for c in range(8):
        chunk = w_col[c * 256:(c + 1) * 256, :]
        jidx = jax.lax.broadcasted_iota(jnp.int32, (256, 1), 0) + c * 256
        gt = chunk > w_row
        eq = chunk == w_row
        stab_row += jnp.sum((gt | (eq & (jidx < i_row))).astype(jnp.int32),
                            axis=0, keepdims=True)
        geq_row += jnp.sum((gt | eq).astype(jnp.int32), axis=0,
                           keepdims=True)
    mask_row = geq_row <= KC
    n_sel = jnp.sum(mask_row.astype(jnp.int32), axis=1, keepdims=True)
    slotc_ref[0] = jnp.where(mask_row, stab_row, KC)

    slots = jax.lax.broadcasted_iota(jnp.int32, (K, S), 0)
    toks = jax.lax.broadcasted_iota(jnp.int32, (K, S), 1)
    sel_onehot = (slots == stab_row) & mask_row
    dup_onehot = (slots >= n_sel) & (slots < KC) & (toks == S - 1)
    p = jnp.where(sel_onehot | dup_onehot, 1.0, 0.0)

    sel32 = jnp.dot(p.astype(jnp.bfloat16), x_ref[0].astype(jnp.bfloat16),
                    preferred_element_type=jnp.float32)
    sel_ref[0] = sel32.astype(jnp.bfloat16)
    selw_ref[0] = jnp.sum(p * w_row, axis=1, keepdims=True)
    h_ref[0] = _ln_rows(sel32, g_ref[:], b_ref[:]).astype(jnp.bfloat16)


def _qkv_kernel(h_ref, wq_ref, wk_ref, wv_ref, bq_ref, bk_ref, bv_ref,
                q_ref, k_ref, v_ref):
    h = h_ref[:]
    q_ref[:] = (jnp.dot(h, wq_ref[:].astype(jnp.bfloat16),
                        preferred_element_type=jnp.float32)
                + bq_ref[:]).astype(jnp.bfloat16)
    k_ref[:] = (jnp.dot(h, wk_ref[:].astype(jnp.bfloat16),
                        preferred_element_type=jnp.float32)
                + bk_ref[:]).astype(jnp.bfloat16)
    v_ref[:] = (jnp.dot(h, wv_ref[:].astype(jnp.bfloat16),
                        preferred_element_type=jnp.float32)
                + bv_ref[:]).astype(jnp.bfloat16)


_HPG = 4


def _attn_kernel(q_ref, k_ref, v_ref, out_ref):
    key_id = jax.lax.broadcasted_iota(jnp.int32, (K, K), 1)
    pad = key_id >= KC
    for t in range(_HPG):
        ds = pl.ds(t * DH, DH)
        s = jax.lax.dot_general(q_ref[:, ds], k_ref[:, ds],
                                (((1,), (1,)), ((), ())),
                                preferred_element_type=jnp.float32)
        s = s / _SQRT_DH
        s = jnp.where(pad, _NEG, s)
        m = jnp.max(s, axis=-1, keepdims=True)
        e = jnp.exp(s - m)
        a = e / jnp.sum(e, axis=-1, keepdims=True)
        out_ref[:, ds] = jnp.dot(a.astype(jnp.bfloat16), v_ref[:, ds],
                                 preferred_element_type=jnp.float32
                                 ).astype(jnp.bfloat16)


def _oproj_kernel(a_ref, wo_ref, bo_ref, sel_ref, g_ref, b_ref,
                  sel2_ref, h2_ref):
    o = jnp.dot(a_ref[:], wo_ref[:], preferred_element_type=jnp.float32)
    sel2 = sel_ref[:].astype(jnp.float32) + o + bo_ref[:]
    sel2_ref[:] = sel2.astype(jnp.bfloat16)
    h2_ref[:] = _ln_rows(sel2, g_ref[:], b_ref[:]).astype(jnp.bfloat16)


_NKFF = 16


def _mlp_kernel(h2_ref, w1_ref, b1_ref, w2_ref, y_ref):
    kk = pl.program_id(0)
    a = jnp.dot(h2_ref[:], w1_ref[:].astype(jnp.bfloat16),
                preferred_element_type=jnp.float32) + b1_ref[:]
    g = jax.nn.gelu(a).astype(jnp.bfloat16)
    for n in range(4):
        ds = pl.ds(n * 512, 512)
        part = jnp.dot(g, w2_ref[:, ds].astype(jnp.bfloat16),
                       preferred_element_type=jnp.float32)

        @pl.when(kk == 0)
        def _():
            y_ref[:, ds] = part

        @pl.when(kk > 0)
        def _():
            y_ref[:, ds] = y_ref[:, ds] + part


def _combine_kernel(slotc_ref, y_ref, sel2_ref, selw_ref, b2_ref, x_ref,
                    out_ref):
    yf = (sel2_ref[0].astype(jnp.float32) + y_ref[0] + b2_ref[:]
          ) * selw_ref[0]
    yb = yf.astype(jnp.bfloat16)
    slotc = slotc_ref[0]
    slot_id = jax.lax.broadcasted_iota(jnp.int32, (slotc.shape[0], K), 1)
    pt = jnp.where(slot_id == slotc, 1.0, 0.0).astype(jnp.bfloat16)
    ysel = jnp.dot(pt, yb, preferred_element_type=jnp.float32)
    out_ref[0] = jnp.where(slotc < KC, ysel, x_ref[0])


def kernel(x, attention_mask, position_ids, Wr, br, ln1_g, ln1_b,
           Wq, bq, Wk, bk, Wv, bv, Wo, bo, ln2_g, ln2_b, W1, b1, W2, b2):
    del attention_mask, position_ids
    f32, bf16 = jnp.float32, jnp.bfloat16

    w = (x @ Wr + br).squeeze(-1)
    w_row = w.reshape(B, 1, S)
    w_col = w.reshape(B, S, 1)

    g1, b1_, g2, b2_ = (ln1_g.reshape(1, D), ln1_b.reshape(1, D),
                        ln2_g.reshape(1, D), ln2_b.reshape(1, D))

    sel, h, selw, slotc = pl.pallas_call(
        _route_gather_kernel,
        grid=(B,),
        in_specs=[
            pl.BlockSpec((1, S, D), lambda i: (i, 0, 0)),
            pl.BlockSpec((1, 1, S), lambda i: (i, 0, 0)),
            pl.BlockSpec((1, S, 1), lambda i: (i, 0, 0)),
            pl.BlockSpec((1, D), lambda i: (0, 0)),
            pl.BlockSpec((1, D), lambda i: (0, 0)),
        ],
        out_specs=[
            pl.BlockSpec((1, K, D), lambda i: (i, 0, 0)),
            pl.BlockSpec((1, K, D), lambda i: (i, 0, 0)),
            pl.BlockSpec((1, K, 1), lambda i: (i, 0, 0)),
            pl.BlockSpec((1, 1, S), lambda i: (i, 0, 0)),
        ],
        out_shape=[
            jax.ShapeDtypeStruct((B, K, D), bf16),
            jax.ShapeDtypeStruct((B, K, D), bf16),
            jax.ShapeDtypeStruct((B, K, 1), f32),
            jax.ShapeDtypeStruct((B, 1, S), jnp.int32),
        ],
    )(x, w_row, w_col, g1, b1_)
    slotc = slotc.reshape(B, S, 1)

    h2d = h.reshape(T, D)
    sel2d = sel.reshape(T, D)

    BN = 256
    NB = D // BN
    q, kproj, v = pl.pallas_call(
        _qkv_kernel,
        grid=(NB,),
        in_specs=[
            pl.BlockSpec((T, D), lambda j: (0, 0)),
            pl.BlockSpec((D, BN), lambda j: (0, j)),
            pl.BlockSpec((D, BN), lambda j: (0, j)),
            pl.BlockSpec((D, BN), lambda j: (0, j)),
            pl.BlockSpec((1, BN), lambda j: (0, j)),
            pl.BlockSpec((1, BN), lambda j: (0, j)),
            pl.BlockSpec((1, BN), lambda j: (0, j)),
        ],
        out_specs=[
            pl.BlockSpec((T, BN), lambda j: (0, j)),
            pl.BlockSpec((T, BN), lambda j: (0, j)),
            pl.BlockSpec((T, BN), lambda j: (0, j)),
        ],
        out_shape=[
            jax.ShapeDtypeStruct((T, D), bf16),
            jax.ShapeDtypeStruct((T, D), bf16),
            jax.ShapeDtypeStruct((T, D), bf16),
        ],
    )(h2d, Wq, Wk, Wv, bq.reshape(1, D), bk.reshape(1, D), bv.reshape(1, D))

    attn = pl.pallas_call(
        _attn_kernel,
        grid=(B, H // _HPG),
        in_specs=[
            pl.BlockSpec((K, _HPG * DH), lambda b, hh: (b, hh)),
            pl.BlockSpec((K, _HPG * DH), lambda b, hh: (b, hh)),
            pl.BlockSpec((K, _HPG * DH), lambda b, hh: (b, hh)),
        ],
        out_specs=pl.BlockSpec((K, _HPG * DH), lambda b, hh: (b, hh)),
        out_shape=jax.ShapeDtypeStruct((T, D), bf16),
    )(q, kproj, v)

    BM = 512
    wo_b = Wo.astype(bf16)
    sel2, h2 = pl.pallas_call(
        _oproj_kernel,
        grid=(T // BM,),
        in_specs=[
            pl.BlockSpec((BM, D), lambda i: (i, 0)),
            pl.BlockSpec((D, D), lambda i: (0, 0)),
            pl.BlockSpec((1, D), lambda i: (0, 0)),
            pl.BlockSpec((BM, D), lambda i: (i, 0)),
            pl.BlockSpec((1, D), lambda i: (0, 0)),
            pl.BlockSpec((1, D), lambda i: (0, 0)),
        ],
        out_specs=[
            pl.BlockSpec((BM, D), lambda i: (i, 0)),
            pl.BlockSpec((BM, D), lambda i: (i, 0)),
        ],
        out_shape=[
            jax.ShapeDtypeStruct((T, D), bf16),
            jax.ShapeDtypeStruct((T, D), bf16),
        ],
    )(attn, wo_b, bo.reshape(1, D), sel2d, g2, b2_)

    BK = FF // _NKFF
    y = pl.pallas_call(
        _mlp_kernel,
        grid=(_NKFF,),
        in_specs=[
            pl.BlockSpec((T, D), lambda kk: (0, 0)),
            pl.BlockSpec((D, BK), lambda kk: (0, kk)),
            pl.BlockSpec((1, BK), lambda kk: (0, kk)),
            pl.BlockSpec((BK, D), lambda kk: (kk, 0)),
        ],
        out_specs=pl.BlockSpec((T, D), lambda kk: (0, 0)),
        out_shape=jax.ShapeDtypeStruct((T, D), f32),
    )(h2, W1, b1.reshape(1, FF), W2)

    y3 = y.reshape(B, K, D)
    sel23 = sel2.reshape(B, K, D)
    selw3 = selw

    ST = 512
    out = pl.pallas_call(
        _combine_kernel,
        grid=(B, S // ST),
        in_specs=[
            pl.BlockSpec((1, ST, 1), lambda b, t: (b, t, 0)),
            pl.BlockSpec((1, K, D), lambda b, t: (b, 0, 0)),
            pl.BlockSpec((1, K, D), lambda b, t: (b, 0, 0)),
            pl.BlockSpec((1, K, 1), lambda b, t: (b, 0, 0)),
            pl.BlockSpec((1, D), lambda b, t: (0, 0)),
            pl.BlockSpec((1, ST, D), lambda b, t: (b, t, 0)),
        ],
        out_specs=pl.BlockSpec((1, ST, D), lambda b, t: (b, t, 0)),
        out_shape=jax.ShapeDtypeStruct((B, S, D), f32),
    )(slotc, y3, sel23, selw3, b2.reshape(1, D), x)

    return out
